# baseline (speedup 1.0000x reference)
import functools

import jax
import jax.numpy as jnp
from jax import lax
from jax.experimental import pallas as pl
from jax.experimental.pallas import tpu as pltpu
from jax.experimental.pallas import tpu_sc as plsc

D_MODEL = 64
SCALE = float(D_MODEL) ** 0.5
LANES = 16

NC, NS = 2, 16
NW = NC * NS
B_TOTAL = 4096 * 200
PER_W = B_TOTAL // NW
CHUNK = 128
NCHUNK = PER_W // CHUNK
NBUF = 4
ROUNDS = NCHUNK // NBUF


def _tec_body(x_hbm, table_hbm, out_hbm, *sc):
    idx_v = sc[0]
    gbuf = sc[1:1 + NBUF]
    obuf = sc[1 + NBUF:1 + 2 * NBUF]
    gsem = sc[1 + 2 * NBUF:1 + 3 * NBUF]
    osem = sc[1 + 3 * NBUF:1 + 4 * NBUF]

    wid = lax.axis_index("c") * NS + lax.axis_index("s")
    base = wid * PER_W

    pltpu.sync_copy(x_hbm.at[wid], idx_v)

    def start_gather(b, g):
        pltpu.async_copy(table_hbm.at[idx_v.at[g]], gbuf[b], gsem[b])

    def wait_gather(b):
        pltpu.make_async_copy(table_hbm.at[idx_v.at[0]], gbuf[b], gsem[b]).wait()

    def start_out(b, g):
        dst = out_hbm.at[pl.ds(base + g * CHUNK, CHUNK)]
        pltpu.async_copy(obuf[b], dst, osem[b])

    def wait_out(b):
        dst = out_hbm.at[pl.ds(base, CHUNK)]
        pltpu.make_async_copy(obuf[b], dst, osem[b]).wait()

    def scale(b):
        gb, ob = gbuf[b], obuf[b]

        def body_fn(i, carry):
            r0 = i * 4
            for u in range(4):
                for j in range(D_MODEL // LANES):
                    s = pl.ds(j * LANES, LANES)
                    ob[r0 + u, s] = gb[r0 + u, s] * SCALE
            return carry

        lax.fori_loop(0, CHUNK // 4, body_fn, 0)

    for b in range(NBUF):
        start_gather(b, b)

    for b in range(NBUF):
        wait_gather(b)
        scale(b)
        start_gather(b, b + NBUF)
        start_out(b, b)

    def round_body(ro, carry):
        for b in range(NBUF):
            g = ro * NBUF + b
            wait_gather(b)
            wait_out(b)
            scale(b)
            start_gather(b, g + NBUF)
            start_out(b, g)
        return carry

    lax.fori_loop(1, ROUNDS - 1, round_body, 0)

    for b in range(NBUF):
        g = (ROUNDS - 1) * NBUF + b
        wait_gather(b)
        wait_out(b)
        scale(b)
        start_out(b, g)

    for b in range(NBUF):
        wait_out(b)


_emb = functools.partial(
    pl.kernel,
    out_type=jax.ShapeDtypeStruct((B_TOTAL, D_MODEL), jnp.float32),
    mesh=plsc.VectorSubcoreMesh(core_axis_name="c", subcore_axis_name="s"),
    scratch_types=(
        [pltpu.VMEM((NCHUNK, CHUNK), jnp.int32)]
        + [pltpu.VMEM((CHUNK, D_MODEL), jnp.float32) for _ in range(2 * NBUF)]
        + [pltpu.SemaphoreType.DMA for _ in range(2 * NBUF)]
    ),
    compiler_params=pltpu.CompilerParams(use_tc_tiling_on_sc=False),
)(_tec_body)


def kernel(x, table):
    x32 = x.reshape(NW, NCHUNK, CHUNK).astype(jnp.int32)
    out = _emb(x32, table)
    return out.reshape(x.shape[0], x.shape[1], D_MODEL)
